# Initial kernel scaffold; baseline (speedup 1.0000x reference)
import jax
import jax.numpy as jnp
from jax.experimental import pallas as pl


def kernel(att):
    raise NotImplementedError("write your pallas kernel here")


# speedup vs baseline: 19.1868x; 19.1868x over previous
import functools

import jax
import jax.numpy as jnp
from jax.experimental import pallas as pl
from jax.experimental.pallas import tpu as pltpu

_ROWS_PER_BLOCK = 16
_SEARCH_ITERS = 31
_ONE_BITS = 0x3F800000


def _loss_block_kernel(x_ref, out_ref, *, k):
    x = x_ref[...]
    xi = jax.lax.bitcast_convert_type(x, jnp.int32)
    r = x.shape[0]

    lo_a = jnp.zeros((r, 1), jnp.int32)
    hi_a = jnp.full((r, 1), _ONE_BITS, jnp.int32)
    lo_b = jnp.full((r, 1), -1, jnp.int32)
    hi_b = jnp.full((r, 1), _ONE_BITS - 1, jnp.int32)

    def body(_, carry):
        lo_a, hi_a, lo_b, hi_b = carry
        mid_a = (lo_a + hi_a) >> 1
        mid_b = (lo_b + hi_b) >> 1
        cnt_a = jnp.sum((xi >= mid_a).astype(jnp.int32), axis=1, keepdims=True)
        cnt_b = jnp.sum((xi <= mid_b).astype(jnp.int32), axis=1, keepdims=True)
        pred_a = cnt_a >= k
        pred_b = cnt_b >= k
        lo_a = jnp.where(pred_a, mid_a, lo_a)
        hi_a = jnp.where(pred_a, hi_a, mid_a)
        hi_b = jnp.where(pred_b, mid_b, hi_b)
        lo_b = jnp.where(pred_b, lo_b, mid_b)
        return lo_a, hi_a, lo_b, hi_b

    lo_a, hi_a, lo_b, hi_b = jax.lax.fori_loop(
        0, _SEARCH_ITERS, body, (lo_a, hi_a, lo_b, hi_b))

    theta_hi = jax.lax.bitcast_convert_type(lo_a, jnp.float32)
    theta_lo = jax.lax.bitcast_convert_type(hi_b, jnp.float32)

    gt = x > theta_hi
    lt = x < theta_lo
    cnt_gt = jnp.sum(gt.astype(jnp.float32), axis=1, keepdims=True)
    cnt_lt = jnp.sum(lt.astype(jnp.float32), axis=1, keepdims=True)
    sum_gt = jnp.sum(jnp.where(gt, x, 0.0), axis=1, keepdims=True)
    sum_lt = jnp.sum(jnp.where(lt, x, 0.0), axis=1, keepdims=True)
    sum_top = sum_gt + (k - cnt_gt) * theta_hi
    sum_bot = sum_lt + (k - cnt_lt) * theta_lo

    partial = jnp.sum(sum_bot - sum_top)

    @pl.when(pl.program_id(0) == 0)
    def _():
        out_ref[0, 0] = 0.0

    out_ref[0, 0] += partial


def kernel(att):
    n, t = att.shape
    k = max(t // 8, 1)
    grid = n // _ROWS_PER_BLOCK
    out = pl.pallas_call(
        functools.partial(_loss_block_kernel, k=k),
        grid=(grid,),
        in_specs=[pl.BlockSpec((_ROWS_PER_BLOCK, t), lambda i: (i, 0))],
        out_specs=pl.BlockSpec((1, 1), lambda i: (0, 0),
                               memory_space=pltpu.SMEM),
        out_shape=jax.ShapeDtypeStruct((1, 1), jnp.float32),
    )(att)
    return (out[0, 0] / (k * n)).astype(jnp.float32)
